# baseline (speedup 1.0000x reference)
import functools

import jax
import jax.numpy as jnp
from jax import lax
from jax.experimental import pallas as pl
from jax.experimental.pallas import tpu as pltpu
from jax.experimental.pallas import tpu_sc as plsc

_CHUNK = 128


@functools.cache
def _build(B, V, D, NC, NS):
    NW = NC * NS
    b_per_w = B // NW
    n_chunk = b_per_w // _CHUNK

    mesh = plsc.VectorSubcoreMesh(core_axis_name="c", subcore_axis_name="s")

    @functools.partial(
        pl.kernel,
        mesh=mesh,
        out_type=jax.ShapeDtypeStruct((B, D), jnp.float32),
        scratch_types=[
            pltpu.VMEM((n_chunk, _CHUNK), jnp.int32),
            pltpu.VMEM((b_per_w, D), jnp.float32),
            pltpu.SemaphoreType.DMA,
        ],
        compiler_params=pltpu.CompilerParams(
            use_tc_tiling_on_sc=False,
            skip_device_barrier=True,
        ),
    )
    def gather_kernel(idx_hbm, table_hbm, out_hbm, idx_v, rows_v, sem):
        wid = lax.axis_index("s") * NC + lax.axis_index("c")
        pltpu.sync_copy(idx_hbm.at[wid], idx_v)
        for j in range(n_chunk):
            pltpu.async_copy(
                table_hbm.at[idx_v.at[j]],
                rows_v.at[pl.ds(j * _CHUNK, _CHUNK)],
                sem,
            )
        pltpu.make_async_copy(
            table_hbm.at[pl.ds(0, b_per_w)], rows_v, sem
        ).wait()
        pltpu.sync_copy(rows_v, out_hbm.at[pl.ds(wid * b_per_w, b_per_w)])

    return gather_kernel


def kernel(users, user_embed):
    B, = users.shape
    V, D = user_embed.shape
    info = plsc.get_sparse_core_info()
    NC, NS = info.num_cores, info.num_subcores
    NW = NC * NS
    b_per_w = B // NW
    idx = users.astype(jnp.int32).reshape(NW, b_per_w // _CHUNK, _CHUNK)
    return _build(B, V, D, NC, NS)(idx, user_embed)


# speedup vs baseline: 2.1545x; 2.1545x over previous
import functools

import jax
import jax.numpy as jnp
from jax import lax
from jax.experimental import pallas as pl
from jax.experimental.pallas import tpu as pltpu
from jax.experimental.pallas import tpu_sc as plsc

_G = 16


@functools.cache
def _build(B, V, D, NC, NS):
    NW = NC * NS
    n = B // NW
    NGRP = n // _G

    mesh = plsc.VectorSubcoreMesh(core_axis_name="c", subcore_axis_name="s")

    @functools.partial(
        pl.kernel,
        mesh=mesh,
        out_type=jax.ShapeDtypeStruct((B, D), jnp.float32),
        scratch_types=[
            pltpu.VMEM((NGRP, _G), jnp.int32),
            pltpu.VMEM((_G, 8, D), jnp.float32),
            pltpu.VMEM((n, D), jnp.float32),
            pltpu.SemaphoreType.DMA,
        ],
        compiler_params=pltpu.CompilerParams(
            use_tc_tiling_on_sc=True,
            skip_device_barrier=True,
            needs_layout_passes=False,
        ),
    )
    def gather_kernel(users_hbm, table_hbm, out_hbm, uv, tiles_v, rows_v, sem):
        wid = lax.axis_index("s") * NC + lax.axis_index("c")
        pltpu.sync_copy(users_hbm.at[wid], uv)
        lanes = lax.broadcasted_iota(jnp.int32, (_G,), 0)

        def group_body(g, carry):
            vec = uv[g, :]
            for q in range(_G):
                u = jnp.max(jnp.where(lanes == q, vec, 0))
                t = lax.shift_right_logical(u, 3)
                pltpu.async_copy(
                    table_hbm.at[pl.ds(t, 1)], tiles_v.at[pl.ds(q, 1)], sem
                )
            pltpu.make_async_copy(
                table_hbm.at[pl.ds(0, _G)], tiles_v, sem
            ).wait()
            for q in range(_G):
                u = jnp.max(jnp.where(lanes == q, vec, 0))
                r = lax.bitwise_and(u, 7)
                for m in range(D // _G):
                    rows_v[g * _G + q, pl.ds(m * _G, _G)] = tiles_v[
                        q, r, pl.ds(m * _G, _G)
                    ]
            return carry

        lax.fori_loop(0, NGRP, group_body, 0)
        pltpu.sync_copy(rows_v, out_hbm.at[pl.ds(wid * n, n)])

    return gather_kernel


def kernel(users, user_embed):
    B, = users.shape
    V, D = user_embed.shape
    info = plsc.get_sparse_core_info()
    NC, NS = info.num_cores, info.num_subcores
    NW = NC * NS
    n = B // NW
    table3 = user_embed.reshape(V // 8, 8, D)
    u = users.astype(jnp.int32).reshape(NW, n // _G, _G)
    return _build(B, V, D, NC, NS)(u, table3)
